# baseline (speedup 1.0000x reference)
import functools

import jax
import jax.numpy as jnp
from jax import lax
from jax.experimental import pallas as pl
from jax.experimental.pallas import tpu as pltpu
from jax.experimental.pallas import tpu_sc as plsc

NUM_PATCHES = 1024
PROJ_DIM = 768
BATCH = 64

NUM_CORES = 2
NUM_SUBCORES = 16
NW = NUM_CORES * NUM_SUBCORES
ROWS_PER_W = NUM_PATCHES // NW
LANES = 16
COL_CHUNKS = PROJ_DIM // LANES


def _sc_body(patches_hbm, pos_hbm, out_hbm, pos_v, in0, in1, ou0, ou1,
             si0, si1, so0, so1):
    wid = lax.axis_index("s") * NUM_CORES + lax.axis_index("c")
    base = wid * ROWS_PER_W
    rows = pl.ds(base, ROWS_PER_W)
    pltpu.sync_copy(pos_hbm.at[rows], pos_v)

    ins = [in0, in1]
    outs = [ou0, ou1]
    sis = [si0, si1]
    sos = [so0, so1]

    pltpu.async_copy(patches_hbm.at[0, rows], in0, si0)
    pltpu.async_copy(patches_hbm.at[1, rows], in1, si1)

    @pl.loop(0, BATCH, step=2)
    def _batch(b):
        for k in range(2):
            i = b + k
            pltpu.make_async_copy(patches_hbm.at[i, rows], ins[k], sis[k]).wait()

            @pl.when(i >= 2)
            def _wait_store():
                pltpu.make_async_copy(outs[k], out_hbm.at[i - 2, rows],
                                      sos[k]).wait()

            @pl.loop(0, ROWS_PER_W)
            def _row(r):
                for c in range(COL_CHUNKS):
                    sl = pl.ds(c * LANES, LANES)
                    outs[k][r, sl] = ins[k][r, sl] + pos_v[r, sl]

            pltpu.async_copy(outs[k], out_hbm.at[i, rows], sos[k])

            @pl.when(i + 2 < BATCH)
            def _next_load():
                pltpu.async_copy(patches_hbm.at[i + 2, rows], ins[k], sis[k])

    pltpu.make_async_copy(ou0, out_hbm.at[BATCH - 2, rows], so0).wait()
    pltpu.make_async_copy(ou1, out_hbm.at[BATCH - 1, rows], so1).wait()


_sc_kernel = functools.partial(
    pl.kernel,
    out_type=jax.ShapeDtypeStruct((BATCH, NUM_PATCHES, PROJ_DIM), jnp.float32),
    mesh=plsc.VectorSubcoreMesh(core_axis_name="c", subcore_axis_name="s"),
    scratch_types=[
        pltpu.VMEM((ROWS_PER_W, PROJ_DIM), jnp.float32),
        pltpu.VMEM((ROWS_PER_W, PROJ_DIM), jnp.float32),
        pltpu.VMEM((ROWS_PER_W, PROJ_DIM), jnp.float32),
        pltpu.VMEM((ROWS_PER_W, PROJ_DIM), jnp.float32),
        pltpu.VMEM((ROWS_PER_W, PROJ_DIM), jnp.float32),
        pltpu.SemaphoreType.DMA,
        pltpu.SemaphoreType.DMA,
        pltpu.SemaphoreType.DMA,
        pltpu.SemaphoreType.DMA,
    ],
)(_sc_body)


def kernel(patches, pos_table):
    return _sc_kernel(patches, pos_table)
